# baseline (speedup 1.0000x reference)
import jax
import jax.numpy as jnp
from jax import lax
from jax.experimental import pallas as pl
from jax.experimental.pallas import tpu as pltpu

H = 1024
F = 512
E = 8
FS = 1024
T = 2048
NCB = 4
CB = H // NCB


def _silu(v):
    return v * jax.nn.sigmoid(v)


def _gate_shared_body(x_ref, gwt_ref, bias_ref, sg_ref, su_ref,
                      psh_ref, cmb_ref):
    x = x_ref[...]
    logits = jnp.dot(x, gwt_ref[...], preferred_element_type=jnp.float32)
    scores = jax.nn.sigmoid(logits)
    sfc = scores + bias_ref[...]
    colE = lax.broadcasted_iota(jnp.int32, (T, E), 1)
    m1 = jnp.max(sfc, axis=1, keepdims=True)
    i1 = jnp.min(jnp.where(sfc == m1, colE, E), axis=1, keepdims=True)
    oh1 = (colE == i1).astype(jnp.float32)
    sfc2 = jnp.where(oh1 > 0, -jnp.inf, sfc)
    m2 = jnp.max(sfc2, axis=1, keepdims=True)
    i2 = jnp.min(jnp.where(sfc2 == m2, colE, E), axis=1, keepdims=True)
    oh2 = (colE == i2).astype(jnp.float32)
    s1 = jnp.sum(oh1 * scores, axis=1, keepdims=True)
    s2 = jnp.sum(oh2 * scores, axis=1, keepdims=True)
    den = s1 + s2 + 1e-20
    cmb_ref[...] = oh1 * (s1 / den) + oh2 * (s2 / den)

    g = jnp.dot(x, sg_ref[...], preferred_element_type=jnp.float32)
    u = jnp.dot(x, su_ref[...], preferred_element_type=jnp.float32)
    psh_ref[...] = (_silu(g) * u).astype(jnp.bfloat16)


def _gate_shared(x, gate_w, gate_bias, sg, su):
    return pl.pallas_call(
        _gate_shared_body,
        out_shape=(
            jax.ShapeDtypeStruct((T, FS), jnp.bfloat16),
            jax.ShapeDtypeStruct((T, E), jnp.float32),
        ),
    )(x, gate_w.T, gate_bias.reshape(1, E), sg, su)


def _moe_body(x_ref, cmb_ref, psh_ref, w1_ref, w3_ref, w2a_ref, sd_ref,
              o_ref, p_ref):
    i = pl.program_id(0)

    @pl.when(i < E)
    def _():
        x = x_ref[...]
        col = lax.broadcasted_iota(jnp.int32, (T, E), 1)
        ce = jnp.sum(jnp.where(col == i, cmb_ref[...], 0.0), axis=1,
                     keepdims=True)
        h1 = jnp.dot(x, w1_ref[0].astype(jnp.bfloat16),
                     preferred_element_type=jnp.float32)
        h3 = jnp.dot(x, w3_ref[0].astype(jnp.bfloat16),
                     preferred_element_type=jnp.float32)
        p = (ce * (_silu(h1) * h3)).astype(jnp.bfloat16)
        p_ref[:, pl.ds(pl.multiple_of(i * F, F), F)] = p

    @pl.when(i >= E)
    def _():
        w2b = w2a_ref[...].astype(jnp.bfloat16)
        sdb = sd_ref[...].astype(jnp.bfloat16)
        o_ref[...] = (
            jnp.dot(p_ref[...], w2b, preferred_element_type=jnp.float32)
            + jnp.dot(psh_ref[...], sdb, preferred_element_type=jnp.float32))


def _moe(xb, combine, psh, w1, w2, w3, sd):
    w2a = w2.reshape(E * F, H)

    def _pblk(i):
        return jnp.maximum(i - E, 0)

    return pl.pallas_call(
        _moe_body,
        grid=(E + NCB,),
        in_specs=[
            pl.BlockSpec((T, H), lambda i: (0, 0)),
            pl.BlockSpec((T, E), lambda i: (0, 0)),
            pl.BlockSpec((T, FS), lambda i: (0, 0)),
            pl.BlockSpec((1, H, F), lambda i: (jnp.minimum(i, E - 1), 0, 0)),
            pl.BlockSpec((1, H, F), lambda i: (jnp.minimum(i, E - 1), 0, 0)),
            pl.BlockSpec((E * F, CB), lambda i: (0, _pblk(i))),
            pl.BlockSpec((FS, CB), lambda i: (0, _pblk(i))),
        ],
        out_specs=pl.BlockSpec((T, CB), lambda i: (0, _pblk(i))),
        out_shape=jax.ShapeDtypeStruct((T, H), jnp.float32),
        scratch_shapes=[pltpu.VMEM((T, E * F), jnp.bfloat16)],
        compiler_params=pltpu.CompilerParams(
            dimension_semantics=("arbitrary",),
            vmem_limit_bytes=62 * 1024 * 1024,
        ),
    )(xb, combine, psh, w1, w3, w2a, sd)


@jax.jit
def kernel(hidden_states, gate_w, gate_bias, w1, w2, w3, sg, su, sd):
    orig_shape = hidden_states.shape
    x = hidden_states.reshape(T, H)
    psh, combine = _gate_shared(x, gate_w, gate_bias, sg, su)
    out = _moe(x.astype(jnp.bfloat16), combine, psh, w1, w2, w3, sd)
    return out.reshape(orig_shape)


# speedup vs baseline: 1.0297x; 1.0297x over previous
import jax
import jax.numpy as jnp
from jax import lax
from jax.experimental import pallas as pl
from jax.experimental.pallas import tpu as pltpu

H = 1024
F = 512
E = 8
FS = 1024
T = 2048


def _silu(v):
    return v * jax.nn.sigmoid(v)


def _gate_shared_body(x_ref, gwt_ref, bias_ref, sg_ref, su_ref, sd_ref,
                      sh_ref, cmb_ref):
    x = x_ref[...]
    logits = jnp.dot(x, gwt_ref[...], preferred_element_type=jnp.float32)
    scores = jax.nn.sigmoid(logits)
    sfc = scores + bias_ref[...]
    colE = lax.broadcasted_iota(jnp.int32, (T, E), 1)
    m1 = jnp.max(sfc, axis=1, keepdims=True)
    i1 = jnp.min(jnp.where(sfc == m1, colE, E), axis=1, keepdims=True)
    oh1 = (colE == i1).astype(jnp.float32)
    sfc2 = jnp.where(oh1 > 0, -jnp.inf, sfc)
    m2 = jnp.max(sfc2, axis=1, keepdims=True)
    i2 = jnp.min(jnp.where(sfc2 == m2, colE, E), axis=1, keepdims=True)
    oh2 = (colE == i2).astype(jnp.float32)
    s1 = jnp.sum(oh1 * scores, axis=1, keepdims=True)
    s2 = jnp.sum(oh2 * scores, axis=1, keepdims=True)
    den = s1 + s2 + 1e-20
    cmb_ref[...] = oh1 * (s1 / den) + oh2 * (s2 / den)

    g = jnp.dot(x, sg_ref[...], preferred_element_type=jnp.float32)
    u = jnp.dot(x, su_ref[...], preferred_element_type=jnp.float32)
    sh_ref[...] = jnp.dot(_silu(g) * u, sd_ref[...],
                          preferred_element_type=jnp.float32)


def _gate_shared(x, gate_w, gate_bias, sg, su, sd):
    return pl.pallas_call(
        _gate_shared_body,
        out_shape=(
            jax.ShapeDtypeStruct((T, H), jnp.float32),
            jax.ShapeDtypeStruct((T, E), jnp.float32),
        ),
    )(x, gate_w.T, gate_bias.reshape(1, E), sg, su, sd)


NCB = 4
CB = H // NCB


def _moe_body(x_ref, cmb_ref, sh_ref, w1_ref, w3_ref, w2a_ref, o_ref, p_ref):
    i = pl.program_id(0)

    @pl.when(i < E)
    def _():
        x = x_ref[...]
        h1 = jnp.dot(x, w1_ref[0], preferred_element_type=jnp.float32)
        h3 = jnp.dot(x, w3_ref[0], preferred_element_type=jnp.float32)
        col = lax.broadcasted_iota(jnp.int32, (T, E), 1)
        ce = jnp.sum(jnp.where(col == i, cmb_ref[...], 0.0), axis=1,
                     keepdims=True)
        p = (ce * (_silu(h1) * h3)).astype(jnp.bfloat16)
        p_ref[:, pl.ds(pl.multiple_of(i * F, F), F)] = p

    @pl.when(i >= E)
    def _():
        w2b = w2a_ref[...].astype(jnp.bfloat16)
        o_ref[...] = sh_ref[...] + jnp.dot(
            p_ref[...], w2b, preferred_element_type=jnp.float32)


def _moe(x, combine, shared_out, w1, w2, w3):
    w2a = w2.reshape(E * F, H)

    def _pblk(i):
        return jnp.maximum(i - E, 0)

    return pl.pallas_call(
        _moe_body,
        grid=(E + NCB,),
        in_specs=[
            pl.BlockSpec((T, H), lambda i: (0, 0)),
            pl.BlockSpec((T, E), lambda i: (0, 0)),
            pl.BlockSpec((T, CB), lambda i: (0, _pblk(i))),
            pl.BlockSpec((1, H, F), lambda i: (jnp.minimum(i, E - 1), 0, 0)),
            pl.BlockSpec((1, H, F), lambda i: (jnp.minimum(i, E - 1), 0, 0)),
            pl.BlockSpec((E * F, CB), lambda i: (0, _pblk(i))),
        ],
        out_specs=pl.BlockSpec((T, CB), lambda i: (0, _pblk(i))),
        out_shape=jax.ShapeDtypeStruct((T, H), jnp.float32),
        scratch_shapes=[pltpu.VMEM((T, E * F), jnp.bfloat16)],
        compiler_params=pltpu.CompilerParams(
            dimension_semantics=("arbitrary",),
            vmem_limit_bytes=100 * 1024 * 1024,
        ),
    )(x, combine, shared_out, w1, w3, w2a)


@jax.jit
def kernel(hidden_states, gate_w, gate_bias, w1, w2, w3, sg, su, sd):
    orig_shape = hidden_states.shape
    x = hidden_states.reshape(T, H)
    sh, combine = _gate_shared(x, gate_w, gate_bias, sg, su, sd)
    out = _moe(x, combine, sh, w1, w2, w3)
    return out.reshape(orig_shape)
